# speedup vs baseline: 4.8426x; 4.8426x over previous
import functools

import jax
import jax.numpy as jnp
from jax import lax
from jax.experimental import pallas as pl
from jax.experimental.pallas import tpu as pltpu
from jax.experimental.pallas import tpu_sc as plsc

_LANES = 16


def _pooled_lookup(S, SL, V, D):
    info = plsc.get_sparse_core_info()
    NC, NS = info.num_cores, info.num_subcores
    NW = NC * NS
    assert S % NW == 0
    seg_per_w = S // NW
    C = 4
    assert C * SL <= 128
    assert seg_per_w % C == 0
    chunks = seg_per_w // C
    idx_per_chunk = C * SL
    assert (idx_per_chunk % 8) == 0
    n_vreg = D // _LANES

    mesh = plsc.VectorSubcoreMesh(core_axis_name="c", subcore_axis_name="s")

    @functools.partial(
        pl.kernel,
        mesh=mesh,
        out_type=jax.ShapeDtypeStruct((S, D), jnp.float32),
        scratch_types=[
            pltpu.VMEM((idx_per_chunk,), jnp.int32),
            pltpu.VMEM((idx_per_chunk, D), jnp.float32),
            pltpu.VMEM((C, D), jnp.float32),
            pltpu.SemaphoreType.DMA,
        ],
    )
    def k(idx_hbm, table_hbm, out_hbm, idx_v, rows_v, out_v, sem):
        cid = lax.axis_index("c")
        sid = lax.axis_index("s")
        wid = sid * NC + cid
        seg_base = wid * seg_per_w

        def chunk_body(ci, carry):
            base = seg_base + ci * C
            pltpu.sync_copy(idx_hbm.at[pl.ds(base * SL, idx_per_chunk)], idx_v)
            pltpu.async_copy(table_hbm.at[idx_v], rows_v, sem).wait()
            for s in range(C):
                for v in range(n_vreg):
                    acc = rows_v[s * SL, pl.ds(v * _LANES, _LANES)]
                    for j in range(1, SL):
                        acc = acc + rows_v[s * SL + j, pl.ds(v * _LANES, _LANES)]
                    out_v[s, pl.ds(v * _LANES, _LANES)] = acc
            pltpu.sync_copy(out_v, out_hbm.at[pl.ds(base, C)])
            return carry

        lax.fori_loop(0, chunks, chunk_body, 0)

    return k


def kernel(sentences, table):
    B, T, SL = sentences.shape
    V, D = table.shape
    S = B * T
    idx_flat = sentences.reshape(S * SL).astype(jnp.int32)
    k = _pooled_lookup(S, SL, V, D)
    out_flat = k(idx_flat, table)
    return out_flat.reshape(B, T, D)


# speedup vs baseline: 9.8492x; 2.0339x over previous
import functools

import jax
import jax.numpy as jnp
from jax import lax
from jax.experimental import pallas as pl
from jax.experimental.pallas import tpu as pltpu
from jax.experimental.pallas import tpu_sc as plsc

_LANES = 16


def _pooled_lookup(S, SL, V, D):
    info = plsc.get_sparse_core_info()
    NC, NS = info.num_cores, info.num_subcores
    NW = NC * NS
    assert S % NW == 0
    seg_per_w = S // NW
    CSEG = 16
    IDXC = CSEG * SL
    assert seg_per_w % (2 * CSEG) == 0
    chunks = seg_per_w // CSEG
    n_vreg = D // _LANES
    idx_words = seg_per_w * SL
    assert idx_words % 8 == 0 and IDXC % 8 == 0

    pieces = []
    off = 0
    while off < IDXC:
        n = min(128, IDXC - off)
        pieces.append((off, n))
        off += n

    mesh = plsc.VectorSubcoreMesh(core_axis_name="c", subcore_axis_name="s")

    @functools.partial(
        pl.kernel,
        mesh=mesh,
        out_type=jax.ShapeDtypeStruct((S, D), jnp.float32),
        scratch_types=[
            pltpu.VMEM((idx_words,), jnp.int32),
            pltpu.VMEM((IDXC, D), jnp.float32),
            pltpu.VMEM((IDXC, D), jnp.float32),
            pltpu.VMEM((CSEG, D), jnp.float32),
            pltpu.VMEM((CSEG, D), jnp.float32),
            pltpu.SemaphoreType.DMA,
            pltpu.SemaphoreType.DMA,
            pltpu.SemaphoreType.DMA,
            pltpu.SemaphoreType.DMA,
        ],
    )
    def k(idx_hbm, table_hbm, out_hbm, idx_v, rows_a, rows_b, out_a, out_b,
          sem_a, sem_b, sem_oa, sem_ob):
        cid = lax.axis_index("c")
        sid = lax.axis_index("s")
        wid = sid * NC + cid
        seg_base = wid * seg_per_w
        pltpu.sync_copy(idx_hbm.at[pl.ds(seg_base * SL, idx_words)], idx_v)

        def fire(g, rows, sem):
            for (o, n) in pieces:
                pltpu.async_copy(
                    table_hbm.at[idx_v.at[pl.ds(g * IDXC + o, n)]],
                    rows.at[pl.ds(o, n)],
                    sem,
                )

        def drain_rows(rows, sem):
            pltpu.make_async_copy(
                table_hbm.at[pl.ds(0, IDXC)], rows, sem).wait()

        def drain_out(outb, sem):
            pltpu.make_async_copy(
                outb, out_hbm.at[pl.ds(0, CSEG)], sem).wait()

        def reduce(rows, outb):
            def seg_body(s, carry):
                for v in range(n_vreg):
                    acc = rows[s * SL, pl.ds(v * _LANES, _LANES)]
                    for j in range(1, SL):
                        acc = acc + rows[s * SL + j, pl.ds(v * _LANES, _LANES)]
                    outb[s, pl.ds(v * _LANES, _LANES)] = acc
                return carry

            lax.fori_loop(0, CSEG, seg_body, 0)

        def half(i, g, rows, sem, outb, sem_o):
            drain_rows(rows, sem)

            @pl.when(i > 0)
            def _():
                drain_out(outb, sem_o)

            reduce(rows, outb)
            pltpu.async_copy(
                outb, out_hbm.at[pl.ds(seg_base + g * CSEG, CSEG)], sem_o)

            @pl.when(g + 2 < chunks)
            def _():
                fire(g + 2, rows, sem)

        def body(i, carry):
            half(i, 2 * i, rows_a, sem_a, out_a, sem_oa)
            half(i, 2 * i + 1, rows_b, sem_b, out_b, sem_ob)
            return carry

        fire(0, rows_a, sem_a)
        fire(1, rows_b, sem_b)
        lax.fori_loop(0, chunks // 2, body, 0)
        drain_out(out_a, sem_oa)
        drain_out(out_b, sem_ob)

    return k


def kernel(sentences, table):
    B, T, SL = sentences.shape
    V, D = table.shape
    S = B * T
    idx_flat = sentences.reshape(S * SL).astype(jnp.int32)
    k = _pooled_lookup(S, SL, V, D)
    out_flat = k(idx_flat, table)
    return out_flat.reshape(B, T, D)
